# initial kernel scaffold (unmeasured)
import jax
import jax.numpy as jnp
from jax import lax
from jax.experimental import pallas as pl
from jax.experimental.pallas import tpu as pltpu

N_DEV = 8
M_PER = 128
N_COLS = 1024

_RING = (0, 1, 2, 3, 7, 6, 5, 4)


def kernel(x, w_mat):
    m, k_per = x.shape
    _, n = w_mat.shape

    def body(x_ref, w_ref, out_ref, p_ref, stage_ref, comm_ref,
             send_sems, recv_sems):
        ring = jnp.array(_RING, dtype=jnp.int32)
        my = lax.axis_index("i")
        my_pos = ring[my]
        right = ring[(my_pos + 1) % N_DEV]
        left = ring[(my_pos - 1) % N_DEV]

        barrier_sem = pltpu.get_barrier_semaphore()
        for nbr in (left, right):
            pl.semaphore_signal(
                barrier_sem, inc=1,
                device_id=(nbr,), device_id_type=pl.DeviceIdType.MESH,
            )
        pl.semaphore_wait(barrier_sem, 2)

        p_ref[...] = jnp.dot(
            x_ref[...], w_ref[...], preferred_element_type=jnp.float32
        )

        for s in range(N_DEV - 1):
            c_send = ring[(my_pos - 1 - s) % N_DEV]
            if s == 0:
                stage_ref[...] = p_ref[pl.ds(c_send * M_PER, M_PER), :]
                src = stage_ref
            else:
                src = comm_ref.at[s - 1]
            rdma = pltpu.make_async_remote_copy(
                src_ref=src,
                dst_ref=comm_ref.at[s],
                send_sem=send_sems.at[s],
                recv_sem=recv_sems.at[s],
                device_id=(right,),
                device_id_type=pl.DeviceIdType.MESH,
            )
            rdma.start()
            rdma.wait()

            c_recv = ring[(my_pos - 2 - s) % N_DEV]
            local = p_ref[pl.ds(c_recv * M_PER, M_PER), :]
            if s == N_DEV - 2:
                out_ref[...] = comm_ref[s] + local
            else:
                comm_ref[s] = comm_ref[s] + local

    return pl.pallas_call(
        body,
        out_shape=jax.ShapeDtypeStruct((M_PER, N_COLS), jnp.float32),
        in_specs=[
            pl.BlockSpec(memory_space=pltpu.VMEM),
            pl.BlockSpec(memory_space=pltpu.VMEM),
        ],
        out_specs=pl.BlockSpec(memory_space=pltpu.VMEM),
        scratch_shapes=[
            pltpu.VMEM((m, n), jnp.float32),
            pltpu.VMEM((M_PER, N_COLS), jnp.float32),
            pltpu.VMEM((N_DEV - 1, M_PER, N_COLS), jnp.float32),
            pltpu.SemaphoreType.DMA((N_DEV - 1,)),
            pltpu.SemaphoreType.DMA((N_DEV - 1,)),
        ],
        compiler_params=pltpu.CompilerParams(collective_id=0),
    )(x, w_mat)


# baseline (device time: 58695 ns/iter reference)
import jax
import jax.numpy as jnp
from jax import lax
from jax.experimental import pallas as pl
from jax.experimental.pallas import tpu as pltpu

N_DEV = 8
M_PER = 128
N_COLS = 1024


def _ring(v):
    return jnp.where(v < 4, v, 11 - v)


def kernel(x, w_mat):
    m, k_per = x.shape
    _, n = w_mat.shape

    def body(x_ref, w_ref, out_ref, p_ref, stage_ref, comm_ref,
             send_sems, recv_sems):
        my = lax.axis_index("i")
        my_pos = _ring(my)
        right = _ring((my_pos + 1) % N_DEV)
        left = _ring((my_pos - 1) % N_DEV)

        barrier_sem = pltpu.get_barrier_semaphore()
        for nbr in (left, right):
            pl.semaphore_signal(
                barrier_sem, inc=1,
                device_id=(nbr,), device_id_type=pl.DeviceIdType.MESH,
            )
        pl.semaphore_wait(barrier_sem, 2)

        p_ref[...] = jnp.dot(
            x_ref[...], w_ref[...], preferred_element_type=jnp.float32
        )

        for s in range(N_DEV - 1):
            c_send = _ring((my_pos - 1 - s) % N_DEV)
            if s == 0:
                stage_ref[...] = p_ref[pl.ds(c_send * M_PER, M_PER), :]
                src = stage_ref
            else:
                src = comm_ref.at[s - 1]
            rdma = pltpu.make_async_remote_copy(
                src_ref=src,
                dst_ref=comm_ref.at[s],
                send_sem=send_sems.at[s],
                recv_sem=recv_sems.at[s],
                device_id=(right,),
                device_id_type=pl.DeviceIdType.MESH,
            )
            rdma.start()
            rdma.wait()

            c_recv = _ring((my_pos - 2 - s) % N_DEV)
            local = p_ref[pl.ds(c_recv * M_PER, M_PER), :]
            if s == N_DEV - 2:
                out_ref[...] = comm_ref[s] + local
            else:
                comm_ref[s] = comm_ref[s] + local

    return pl.pallas_call(
        body,
        out_shape=jax.ShapeDtypeStruct((M_PER, N_COLS), jnp.float32),
        in_specs=[
            pl.BlockSpec(memory_space=pltpu.VMEM),
            pl.BlockSpec(memory_space=pltpu.VMEM),
        ],
        out_specs=pl.BlockSpec(memory_space=pltpu.VMEM),
        scratch_shapes=[
            pltpu.VMEM((m, n), jnp.float32),
            pltpu.VMEM((M_PER, N_COLS), jnp.float32),
            pltpu.VMEM((N_DEV - 1, M_PER, N_COLS), jnp.float32),
            pltpu.SemaphoreType.DMA((N_DEV - 1,)),
            pltpu.SemaphoreType.DMA((N_DEV - 1,)),
        ],
        compiler_params=pltpu.CompilerParams(collective_id=0),
    )(x, w_mat)


# device time: 32556 ns/iter; 1.8029x vs baseline; 1.8029x over previous
import jax
import jax.numpy as jnp
from jax import lax
from jax.experimental import pallas as pl
from jax.experimental.pallas import tpu as pltpu

N_DEV = 8
M_PER = 128
N_COLS = 1024


def kernel(x, w_mat):
    m, k_per = x.shape
    _, n = w_mat.shape

    def body(x_ref, w_ref, out_ref, p_ref,
             stage_z, stage_y, stage_x, recv_z, recv_y, recv_x,
             send_sems, recv_sems):
        my = lax.axis_index("i")
        q = my % 4
        my_z = my // 4
        my_y = jnp.where(q >= 2, 1, 0)
        pz = my ^ 4
        py = my - q + (3 - q)
        px = my - q + (q ^ 1)

        barrier_sem = pltpu.get_barrier_semaphore()
        for nbr in (pz, py, px):
            pl.semaphore_signal(
                barrier_sem, inc=1,
                device_id=(nbr,), device_id_type=pl.DeviceIdType.MESH,
            )
        pl.semaphore_wait(barrier_sem, 3)

        p_ref[...] = jnp.dot(
            x_ref[...].astype(jnp.bfloat16),
            w_ref[...].astype(jnp.bfloat16),
            preferred_element_type=jnp.float32,
        )

        def exchange(phase, partner, send_row0, nrows, my_row0,
                     stage_ref, recv_ref):
            stage_ref[...] = p_ref[pl.ds(send_row0, nrows), :].astype(
                jnp.bfloat16
            )
            rdma = pltpu.make_async_remote_copy(
                src_ref=stage_ref,
                dst_ref=recv_ref,
                send_sem=send_sems.at[phase],
                recv_sem=recv_sems.at[phase],
                device_id=(partner,),
                device_id_type=pl.DeviceIdType.MESH,
            )
            rdma.start()
            rdma.wait()
            acc = p_ref[pl.ds(my_row0, nrows), :] + recv_ref[...].astype(
                jnp.float32
            )
            return acc

        my_lo_z = my_z * 4 * M_PER
        other_lo_z = (1 - my_z) * 4 * M_PER
        acc = exchange(0, pz, other_lo_z, 4 * M_PER, my_lo_z, stage_z, recv_z)
        p_ref[pl.ds(my_lo_z, 4 * M_PER), :] = acc

        my_lo_y = (my_z * 4 + 2 * my_y) * M_PER
        other_lo_y = (my_z * 4 + 2 * (1 - my_y)) * M_PER
        acc = exchange(1, py, other_lo_y, 2 * M_PER, my_lo_y, stage_y, recv_y)
        p_ref[pl.ds(my_lo_y, 2 * M_PER), :] = acc

        out_ref[...] = exchange(2, px, px * M_PER, M_PER, my * M_PER,
                                stage_x, recv_x)

    return pl.pallas_call(
        body,
        out_shape=jax.ShapeDtypeStruct((M_PER, N_COLS), jnp.float32),
        in_specs=[
            pl.BlockSpec(memory_space=pltpu.VMEM),
            pl.BlockSpec(memory_space=pltpu.VMEM),
        ],
        out_specs=pl.BlockSpec(memory_space=pltpu.VMEM),
        scratch_shapes=[
            pltpu.VMEM((m, n), jnp.float32),
            pltpu.VMEM((4 * M_PER, N_COLS), jnp.bfloat16),
            pltpu.VMEM((2 * M_PER, N_COLS), jnp.bfloat16),
            pltpu.VMEM((M_PER, N_COLS), jnp.bfloat16),
            pltpu.VMEM((4 * M_PER, N_COLS), jnp.bfloat16),
            pltpu.VMEM((2 * M_PER, N_COLS), jnp.bfloat16),
            pltpu.VMEM((M_PER, N_COLS), jnp.bfloat16),
            pltpu.SemaphoreType.DMA((3,)),
            pltpu.SemaphoreType.DMA((3,)),
        ],
        compiler_params=pltpu.CompilerParams(collective_id=0),
    )(x, w_mat)


# device time: 19740 ns/iter; 2.9734x vs baseline; 1.6492x over previous
import jax
import jax.numpy as jnp
from jax import lax
from jax.experimental import pallas as pl
from jax.experimental.pallas import tpu as pltpu

N_DEV = 8
M_PER = 128
M = 1024
N_COLS = 1024

_GROUPS = ((0, 384), (384, 384), (768, 256))


def _perm(order):
    out = []
    for np_ in range(8):
        bits = {a: (np_ >> (2 - i)) & 1 for i, a in enumerate(order)}
        x, y, z = bits["x"], bits["y"], bits["z"]
        out.append(4 * z + 2 * y + (x ^ y))
    return tuple(out)


_ORDERS = (("z", "y", "x"), ("y", "x", "z"), ("x", "z", "y"))
_PERMS = tuple(_perm(o) for o in _ORDERS)


def kernel(x, w_mat):
    def body(x_ref, w_ref, out_ref,
             p0, p1, p2, xg0, xg1, xg2, st0, st1, st2,
             r00, r01, r02, r10, r11, r12, r20, r21, r22,
             send_sems, recv_sems):
        p_refs = (p0, p1, p2)
        xg_refs = (xg0, xg1, xg2)
        st_refs = (st0, st1, st2)
        rv_refs = ((r00, r01, r02), (r10, r11, r12), (r20, r21, r22))

        my = lax.axis_index("i")
        q = my % 4
        my_z = my // 4
        my_y = jnp.where(q >= 2, 1, 0)
        my_x = jnp.where((q == 1) | (q == 2), 1, 0)
        pz = my ^ 4
        py = my - q + (3 - q)
        px = my - q + (q ^ 1)

        coord = {"x": my_x, "y": my_y, "z": my_z}
        partner = {"x": px, "y": py, "z": pz}

        barrier_sem = pltpu.get_barrier_semaphore()
        for nbr in (pz, py, px):
            pl.semaphore_signal(
                barrier_sem, inc=1,
                device_id=(nbr,), device_id_type=pl.DeviceIdType.MESH,
            )
        pl.semaphore_wait(barrier_sem, 3)

        for g, (c0, nc) in enumerate(_GROUPS):
            for np_, cid in enumerate(_PERMS[g]):
                xg_refs[g][np_ * M_PER:(np_ + 1) * M_PER, :] = (
                    x_ref[cid * M_PER:(cid + 1) * M_PER, :]
                    .astype(jnp.bfloat16)
                )
            p_refs[g][...] = jnp.dot(
                xg_refs[g][...],
                w_ref[:, c0:c0 + nc].astype(jnp.bfloat16),
                preferred_element_type=jnp.float32,
            )

        def issue(g, p):
            order = _ORDERS[g]
            nrows = 512 >> p
            prefix = 0
            for j in range(p):
                prefix = prefix + coord[order[j]] * (512 >> j)
            c_p = coord[order[p]]
            send_row0 = prefix + (1 - c_p) * nrows
            keep_row0 = prefix + c_p * nrows
            st_refs[g][pl.ds(0, nrows), :] = (
                p_refs[g][pl.ds(send_row0, nrows), :].astype(jnp.bfloat16)
            )
            rdma = pltpu.make_async_remote_copy(
                src_ref=st_refs[g].at[pl.ds(0, nrows), :],
                dst_ref=rv_refs[g][p],
                send_sem=send_sems.at[g, p],
                recv_sem=recv_sems.at[g, p],
                device_id=(partner[order[p]],),
                device_id_type=pl.DeviceIdType.MESH,
            )
            rdma.start()
            return rdma, keep_row0, nrows

        inflight = [issue(g, 0) for g in range(3)]
        for p in range(1, 3):
            for g in range(3):
                rdma, keep0, nrows = inflight[g]
                rdma.wait()
                p_refs[g][pl.ds(keep0, nrows), :] = (
                    p_refs[g][pl.ds(keep0, nrows), :]
                    + rv_refs[g][p - 1][...].astype(jnp.float32)
                )
                inflight[g] = issue(g, p)
        for g, (c0, nc) in enumerate(_GROUPS):
            rdma, keep0, nrows = inflight[g]
            rdma.wait()
            out_ref[:, c0:c0 + nc] = (
                p_refs[g][pl.ds(keep0, M_PER), :]
                + rv_refs[g][2][...].astype(jnp.float32)
            )

    bf = jnp.bfloat16
    scratch = [
        pltpu.VMEM((M, 384), jnp.float32),
        pltpu.VMEM((M, 384), jnp.float32),
        pltpu.VMEM((M, 256), jnp.float32),
        pltpu.VMEM((M, 128), bf),
        pltpu.VMEM((M, 128), bf),
        pltpu.VMEM((M, 128), bf),
        pltpu.VMEM((512, 384), bf),
        pltpu.VMEM((512, 384), bf),
        pltpu.VMEM((512, 256), bf),
        pltpu.VMEM((512, 384), bf),
        pltpu.VMEM((256, 384), bf),
        pltpu.VMEM((128, 384), bf),
        pltpu.VMEM((512, 384), bf),
        pltpu.VMEM((256, 384), bf),
        pltpu.VMEM((128, 384), bf),
        pltpu.VMEM((512, 256), bf),
        pltpu.VMEM((256, 256), bf),
        pltpu.VMEM((128, 256), bf),
        pltpu.SemaphoreType.DMA((3, 3)),
        pltpu.SemaphoreType.DMA((3, 3)),
    ]
    return pl.pallas_call(
        body,
        out_shape=jax.ShapeDtypeStruct((M_PER, N_COLS), jnp.float32),
        in_specs=[
            pl.BlockSpec(memory_space=pltpu.VMEM),
            pl.BlockSpec(memory_space=pltpu.VMEM),
        ],
        out_specs=pl.BlockSpec(memory_space=pltpu.VMEM),
        scratch_shapes=scratch,
        compiler_params=pltpu.CompilerParams(collective_id=0),
    )(x, w_mat)
